# baseline (speedup 1.0000x reference)
import functools

import jax
import jax.numpy as jnp
from jax import lax
from jax.experimental import pallas as pl
from jax.experimental.pallas import tpu as pltpu
from jax.experimental.pallas import tpu_sc as plsc

_SELU_ALPHA = 1.6732632423543772
_SELU_SCALE = 1.0507009873554805
_RING = 8
_STRIPES = 5


def _selu(x):
    return _SELU_SCALE * jnp.where(x > 0, x, _SELU_ALPHA * (jnp.exp(x) - 1.0))


def _round_bf16_bits(x):
    u = lax.bitcast_convert_type(x, jnp.uint32)
    return u + jnp.uint32(0x7FFF) + ((u >> 16) & jnp.uint32(1))


def _precompute_body(hs_ref, hv_ref, walo_ref, wahi_ref, wblo_ref, wbhi_ref,
                     wdlo_ref, wdhi_ref, welo_ref, wehi_ref, b1lo_ref,
                     b1hi_ref, alo_ref, ahi_ref, p_ref):
    hs = hs_ref[...]
    hv = hv_ref[...]

    def mm(w1, w2):
        return (jnp.dot(hs, w1[...], preferred_element_type=jnp.float32)
                + jnp.dot(hv, w2[...], preferred_element_type=jnp.float32))

    alo_ref[...] = mm(walo_ref, wblo_ref) + b1lo_ref[...]
    ahi_ref[...] = mm(wahi_ref, wbhi_ref) + b1hi_ref[...]
    p_lo = mm(wdlo_ref, welo_ref)
    p_hi = mm(wdhi_ref, wehi_ref)
    packed = ((_round_bf16_bits(p_lo) >> 16)
              | (_round_bf16_bits(p_hi) & jnp.uint32(0xFFFF0000)))
    p_ref[...] = lax.bitcast_convert_type(packed, jnp.int32)


def _precompute(hs, hv, waT, wbT, wdT, weT, b1, block=1000):
    n, c = hs.shape
    h = c // 2
    grid = n // block
    row_spec = pl.BlockSpec((block, c), lambda i: (i, 0))
    half_spec = pl.BlockSpec((block, h), lambda i: (i, 0))
    wh_spec = pl.BlockSpec((c, h), lambda i: (0, 0))
    bh_spec = pl.BlockSpec((1, h), lambda i: (0, 0))
    return pl.pallas_call(
        _precompute_body,
        grid=(grid,),
        in_specs=[row_spec, row_spec] + [wh_spec] * 8 + [bh_spec] * 2,
        out_specs=[half_spec, half_spec, half_spec],
        out_shape=[jax.ShapeDtypeStruct((n, h), jnp.float32),
                   jax.ShapeDtypeStruct((n, h), jnp.float32),
                   jax.ShapeDtypeStruct((n, h), jnp.int32)],
    )(hs, hv, waT[:, :h], waT[:, h:], wbT[:, :h], wbT[:, h:],
      wdT[:, :h], wdT[:, h:], weT[:, :h], weT[:, h:],
      b1[:, :h], b1[:, h:])


def _sc_gather(p_rows, idx2d):
    nchunk, lane = idx2d.shape
    w = p_rows.shape[1]
    info = plsc.get_sparse_core_info()
    nw = info.num_cores * info.num_subcores
    cpw = nchunk // nw
    mesh = plsc.VectorSubcoreMesh(core_axis_name="c", subcore_axis_name="s")
    ring = _RING

    @functools.partial(
        pl.kernel,
        mesh=mesh,
        compiler_params=pltpu.CompilerParams(use_tc_tiling_on_sc=False),
        out_type=jax.ShapeDtypeStruct((nchunk, lane, w), jnp.int32),
        scratch_types=(
            [pltpu.VMEM((cpw, lane), jnp.int32)]
            + [pltpu.VMEM((lane, w), jnp.int32) for _ in range(ring)]
            + [pltpu.SemaphoreType.DMA for _ in range(2 * ring)]
        ),
    )
    def gather_kernel(p_hbm, idx_hbm, out_hbm, idx_v, *bufs_sems):
        bufs = bufs_sems[:ring]
        gsems = bufs_sems[ring:2 * ring]
        osems = bufs_sems[2 * ring:]
        wid = lax.axis_index("s") * info.num_cores + lax.axis_index("c")
        base = wid * cpw
        pltpu.sync_copy(idx_hbm.at[pl.ds(base, cpw)], idx_v)
        for b in range(ring):
            pltpu.async_copy(p_hbm.at[idx_v.at[b]], bufs[b], gsems[b])

        def group(j, _):
            i0 = j * ring
            for b in range(ring):
                i = i0 + b
                pltpu.make_async_copy(p_hbm.at[idx_v.at[i]], bufs[b],
                                      gsems[b]).wait()
                pltpu.async_copy(bufs[b], out_hbm.at[base + i], osems[b])
                pltpu.make_async_copy(bufs[b], out_hbm.at[base + i],
                                      osems[b]).wait()

                @pl.when(i + ring < cpw)
                def _():
                    pltpu.async_copy(p_hbm.at[idx_v.at[i + ring]], bufs[b],
                                     gsems[b])

            return 0

        lax.fori_loop(0, cpw // ring, group, 0)

    return gather_kernel(p_rows, idx2d)


def _mlp_body(hv_ref, alo_ref, ahi_ref, he_ref, g_ref, m_ref, wclo_ref,
              wchi_ref, w2lo_ref, w2hi_ref, w3_ref, b2_ref, b3_ref, out_ref,
              *, k_nbrs, scale):
    a_lo = alo_ref[...]
    a_hi = ahi_ref[...]
    wc_lo = wclo_ref[...]
    wc_hi = wchi_ref[...]
    w2_lo = w2lo_ref[...]
    w2_hi = w2hi_ref[...]
    w3 = w3_ref[...]
    b2 = b2_ref[...]
    acc = jnp.zeros(out_ref.shape, jnp.float32)
    for k in range(k_nbrs):
        he_k = he_ref[:, k, :]
        gu = lax.bitcast_convert_type(g_ref[:, k, :], jnp.uint32)
        g_lo = lax.bitcast_convert_type(gu << 16, jnp.float32)
        g_hi = lax.bitcast_convert_type(gu & jnp.uint32(0xFFFF0000),
                                        jnp.float32)
        m_k = m_ref[:, k:k + 1]
        t_lo = a_lo + m_k * (
            jnp.dot(he_k, wc_lo, preferred_element_type=jnp.float32) + g_lo)
        t_hi = a_hi + m_k * (
            jnp.dot(he_k, wc_hi, preferred_element_type=jnp.float32) + g_hi)
        h1_lo = _selu(t_lo)
        h1_hi = _selu(t_hi)
        h2 = _selu(
            jnp.dot(h1_lo, w2_lo, preferred_element_type=jnp.float32)
            + jnp.dot(h1_hi, w2_hi, preferred_element_type=jnp.float32) + b2)
        acc = acc + jnp.dot(h2, w3, preferred_element_type=jnp.float32)
    dh = (acc + k_nbrs * b3_ref[...]) * (1.0 / scale)
    r = hv_ref[...] + dh
    mu = jnp.mean(r, axis=-1, keepdims=True)
    var = jnp.mean(jnp.square(r - mu), axis=-1, keepdims=True)
    out_ref[...] = (r - mu) * lax.rsqrt(var + 1e-5)


def _mlp(hv, a_lo, a_hi, he3, g3, mask2, wcT, w2T, w3T, b2, b3, block=400):
    n, c = hv.shape
    h = c // 2
    k_nbrs = he3.shape[1]
    grid = n // block
    row_spec = pl.BlockSpec((block, c), lambda i: (i, 0))
    half_spec = pl.BlockSpec((block, h), lambda i: (i, 0))
    he_spec = pl.BlockSpec((block, k_nbrs, c), lambda i: (i, 0, 0))
    g_spec = pl.BlockSpec((block, k_nbrs, h), lambda i: (i, 0, 0))
    mask_spec = pl.BlockSpec((block, k_nbrs), lambda i: (i, 0))
    wch_spec = pl.BlockSpec((c, h), lambda i: (0, 0))
    w2h_spec = pl.BlockSpec((h, c), lambda i: (0, 0))
    w_spec = pl.BlockSpec((c, c), lambda i: (0, 0))
    b_spec = pl.BlockSpec((1, c), lambda i: (0, 0))
    return pl.pallas_call(
        functools.partial(_mlp_body, k_nbrs=k_nbrs, scale=30.0),
        grid=(grid,),
        in_specs=[row_spec, half_spec, half_spec, he_spec, g_spec, mask_spec,
                  wch_spec, wch_spec, w2h_spec, w2h_spec, w_spec, b_spec,
                  b_spec],
        out_specs=row_spec,
        out_shape=jax.ShapeDtypeStruct((n, c), jnp.float32),
    )(hv, a_lo, a_hi, he3, g3, mask2, wcT[:, :h], wcT[:, h:],
      w2T[:h, :], w2T[h:, :], w3T, b2, b3)


def kernel(h_S, h_V, h_E, E_idx, mask_attend, W1, b1, W2, b2, W3, b3):
    b, n, k_nbrs, c = h_E.shape
    hs = h_S.reshape(n, c)
    hv = h_V.reshape(n, c)
    he3 = h_E.reshape(n, k_nbrs, c)
    mask2 = mask_attend.reshape(n, k_nbrs).astype(jnp.float32)

    w1T = W1.T
    waT, wbT, wcT, wdT, weT = [w1T[i * c:(i + 1) * c] for i in range(5)]
    b1r = b1.reshape(1, c)
    b2r = b2.reshape(1, c)
    b3r = b3.reshape(1, c)

    a_lo, a_hi, p_rows = _precompute(hs, hv, waT, wbT, wdT, weT, b1r)

    lane = 128
    info = plsc.get_sparse_core_info()
    nw = info.num_cores * info.num_subcores
    quantum = lane * nw * _RING
    ns = n // _STRIPES
    es = ns * k_nbrs
    eps = ((es + quantum - 1) // quantum) * quantum
    idx_flat = E_idx.reshape(n * k_nbrs).astype(jnp.int32)

    outs = []
    for s in range(_STRIPES):
        sl = slice(s * ns, (s + 1) * ns)
        idx_s = jnp.pad(idx_flat[s * es:(s + 1) * es], (0, eps - es))
        idx2d = idx_s.reshape(eps // lane, lane)
        g = _sc_gather(p_rows, idx2d)
        g3 = g.reshape(eps // k_nbrs, k_nbrs, c // 2)
        outs.append(_mlp(hv[sl], a_lo[sl], a_hi[sl], he3[sl], g3, mask2[sl],
                         wcT, w2T=W2.T, w3T=W3.T, b2=b2r, b3=b3r))
    out = jnp.concatenate(outs, axis=0)
    return out.reshape(b, n, c)


# speedup vs baseline: 1.3381x; 1.3381x over previous
import functools

import jax
import jax.numpy as jnp
from jax import lax
from jax.experimental import pallas as pl
from jax.experimental.pallas import tpu as pltpu
from jax.experimental.pallas import tpu_sc as plsc

_SELU_ALPHA = 1.6732632423543772
_SELU_SCALE = 1.0507009873554805
_RING = 4
_STRIPES = 5


def _selu(x):
    return _SELU_SCALE * jnp.where(x > 0, x, _SELU_ALPHA * (jnp.exp(x) - 1.0))


def _precompute_body(hs_ref, hv_ref, wa_ref, wb_ref, wd_ref, we_ref, b1_ref,
                     a_ref, p_ref):
    hs = hs_ref[...]
    hv = hv_ref[...]

    def mm(w1, w2):
        return (jnp.dot(hs, w1[...], preferred_element_type=jnp.float32)
                + jnp.dot(hv, w2[...], preferred_element_type=jnp.float32))

    a_ref[...] = mm(wa_ref, wb_ref) + b1_ref[...]
    p_ref[...] = mm(wd_ref, we_ref)


def _precompute(hs, hv, waT, wbT, wdT, weT, b1, block=1000):
    n, c = hs.shape
    grid = n // block
    row_spec = pl.BlockSpec((block, c), lambda i: (i, 0))
    w_spec = pl.BlockSpec((c, c), lambda i: (0, 0))
    b_spec = pl.BlockSpec((1, c), lambda i: (0, 0))
    return pl.pallas_call(
        _precompute_body,
        grid=(grid,),
        in_specs=[row_spec, row_spec] + [w_spec] * 4 + [b_spec],
        out_specs=[row_spec, row_spec],
        out_shape=[jax.ShapeDtypeStruct((n, c), jnp.float32),
                   jax.ShapeDtypeStruct((n, c), jnp.float32)],
    )(hs, hv, waT, wbT, wdT, weT, b1)


_CPW_FAST = 12
_CPW_SLOW = 4


def _sc_gather(p_rows, idx2d, nchunk):
    lane = idx2d.shape[1]
    w = p_rows.shape[1]
    info = plsc.get_sparse_core_info()
    ns = info.num_subcores
    mesh = plsc.VectorSubcoreMesh(core_axis_name="c", subcore_axis_name="s")
    ring = _RING
    n_fast = ns * _CPW_FAST

    @functools.partial(
        pl.kernel,
        mesh=mesh,
        compiler_params=pltpu.CompilerParams(use_tc_tiling_on_sc=False),
        out_type=jax.ShapeDtypeStruct((nchunk, lane, w), jnp.float32),
        scratch_types=(
            [pltpu.VMEM((_CPW_FAST, lane), jnp.int32)]
            + [pltpu.VMEM((lane, w), jnp.float32) for _ in range(ring)]
            + [pltpu.SemaphoreType.DMA for _ in range(2 * ring)]
        ),
    )
    def gather_kernel(p_hbm, idx_hbm, out_hbm, idx_v, *bufs_sems):
        bufs = bufs_sems[:ring]
        gsems = bufs_sems[ring:2 * ring]
        osems = bufs_sems[2 * ring:]
        cid = lax.axis_index("c")
        sid = lax.axis_index("s")
        fast = cid == 0
        base = jnp.where(fast, sid * _CPW_FAST, n_fast + sid * _CPW_SLOW)
        cpw = jnp.where(fast, _CPW_FAST, _CPW_SLOW)

        @pl.when(cpw > 0)
        def _():
            pltpu.sync_copy(idx_hbm.at[pl.ds(base, _CPW_FAST)], idx_v)
            for b in range(ring):
                pltpu.async_copy(p_hbm.at[idx_v.at[b]], bufs[b], gsems[b])

        def group(j, _):
            i0 = j * ring
            for b in range(ring):
                i = i0 + b
                pltpu.make_async_copy(p_hbm.at[idx_v.at[i]], bufs[b],
                                      gsems[b]).wait()
                pltpu.async_copy(bufs[b], out_hbm.at[base + i], osems[b])
                pltpu.make_async_copy(bufs[b], out_hbm.at[base + i],
                                      osems[b]).wait()

                @pl.when(i + ring < cpw)
                def _():
                    pltpu.async_copy(p_hbm.at[idx_v.at[i + ring]], bufs[b],
                                     gsems[b])

            return 0

        lax.fori_loop(0, cpw // ring, group, 0)

    return gather_kernel(p_rows, idx2d)


def _mlp_body(hv_ref, a_ref, he_ref, g_ref, m_ref, wc_ref, w2_ref, w3_ref,
              b2_ref, b3_ref, out_ref, *, k_nbrs, scale):
    a = a_ref[...]
    wc = wc_ref[...]
    w2 = w2_ref[...]
    w3 = w3_ref[...]
    b2 = b2_ref[...]
    acc = jnp.zeros(out_ref.shape, jnp.float32)
    for k in range(k_nbrs):
        he_k = he_ref[:, k, :]
        g_k = g_ref[:, k, :]
        m_k = m_ref[:, k:k + 1]
        t = a + m_k * (
            jnp.dot(he_k, wc, preferred_element_type=jnp.float32) + g_k)
        h1 = _selu(t)
        h2 = _selu(jnp.dot(h1, w2, preferred_element_type=jnp.float32) + b2)
        acc = acc + jnp.dot(h2, w3, preferred_element_type=jnp.float32)
    dh = (acc + k_nbrs * b3_ref[...]) * (1.0 / scale)
    r = hv_ref[...] + dh
    mu = jnp.mean(r, axis=-1, keepdims=True)
    var = jnp.mean(jnp.square(r - mu), axis=-1, keepdims=True)
    out_ref[...] = (r - mu) * lax.rsqrt(var + 1e-5)


def _mlp(hv, a, he3, g3, mask2, wcT, w2T, w3T, b2, b3, block=400):
    n, c = hv.shape
    k_nbrs = he3.shape[1]
    grid = n // block
    row_spec = pl.BlockSpec((block, c), lambda i: (i, 0))
    he_spec = pl.BlockSpec((block, k_nbrs, c), lambda i: (i, 0, 0))
    mask_spec = pl.BlockSpec((block, k_nbrs), lambda i: (i, 0))
    w_spec = pl.BlockSpec((c, c), lambda i: (0, 0))
    b_spec = pl.BlockSpec((1, c), lambda i: (0, 0))
    return pl.pallas_call(
        functools.partial(_mlp_body, k_nbrs=k_nbrs, scale=30.0),
        grid=(grid,),
        in_specs=[row_spec, row_spec, he_spec, he_spec, mask_spec,
                  w_spec, w_spec, w_spec, b_spec, b_spec],
        out_specs=row_spec,
        out_shape=jax.ShapeDtypeStruct((n, c), jnp.float32),
    )(hv, a, he3, g3, mask2, wcT, w2T, w3T, b2, b3)


def kernel(h_S, h_V, h_E, E_idx, mask_attend, W1, b1, W2, b2, W3, b3):
    b, n, k_nbrs, c = h_E.shape
    hs = h_S.reshape(n, c)
    hv = h_V.reshape(n, c)
    he3 = h_E.reshape(n, k_nbrs, c)
    mask2 = mask_attend.reshape(n, k_nbrs).astype(jnp.float32)

    w1T = W1.T
    waT, wbT, wcT, wdT, weT = [w1T[i * c:(i + 1) * c] for i in range(5)]
    b1r = b1.reshape(1, c)
    b2r = b2.reshape(1, c)
    b3r = b3.reshape(1, c)

    a_rows, p_rows = _precompute(hs, hv, waT, wbT, wdT, weT, b1r)

    lane = 128
    info = plsc.get_sparse_core_info()
    ns_sub = info.num_subcores
    chunks_stripe = ns_sub * (_CPW_FAST + _CPW_SLOW)
    ns = n // _STRIPES
    es = ns * k_nbrs
    eps = chunks_stripe * lane
    assert eps >= es
    pad_rows = _CPW_FAST - _CPW_SLOW
    idx_flat = E_idx.reshape(n * k_nbrs).astype(jnp.int32)

    outs = []
    for s in range(_STRIPES):
        sl = slice(s * ns, (s + 1) * ns)
        idx_s = jnp.pad(idx_flat[s * es:(s + 1) * es],
                        (0, eps - es + pad_rows * lane))
        idx2d = idx_s.reshape(chunks_stripe + pad_rows, lane)
        g = _sc_gather(p_rows, idx2d, chunks_stripe)
        g3 = g.reshape(eps // k_nbrs, k_nbrs, c)
        outs.append(_mlp(hv[sl], a_rows[sl], he3[sl], g3, mask2[sl],
                         wcT, w2T=W2.T, w3T=W3.T, b2=b2r, b3=b3r))
    out = jnp.concatenate(outs, axis=0)
    return out.reshape(b, n, c)
